# Initial kernel scaffold; baseline (speedup 1.0000x reference)
import jax
import jax.numpy as jnp
from jax.experimental import pallas as pl


def kernel(prev, cur, dim, idx, inp_seq_len):
    raise NotImplementedError("write your pallas kernel here")


# speedup vs baseline: 1.5562x; 1.5562x over previous
import jax
import jax.numpy as jnp
from jax.experimental import pallas as pl
from jax.experimental.pallas import tpu as pltpu


def _copy_scatter_kernel(idx_ref, prev_ref, cur_ref, out_ref):
    out_ref[...] = prev_ref[...]
    q_tot = cur_ref.shape[1]

    def body(q, carry):
        p = idx_ref[q]
        out_ref[0, pl.ds(p, 1), :] = cur_ref[0, pl.ds(q, 1), :]
        return carry

    jax.lax.fori_loop(0, q_tot, body, 0, unroll=True)


def kernel(prev, cur, dim, idx, inp_seq_len):
    B, H, KV, D = prev.shape
    Q = cur.shape[2]
    idx = (idx + (jnp.asarray(dim, dtype=idx.dtype) - 2)).astype(jnp.int32)

    prev3 = prev.reshape(B * H, KV, D)
    cur3 = cur.reshape(B * H, Q, D)

    grid_spec = pltpu.PrefetchScalarGridSpec(
        num_scalar_prefetch=1,
        grid=(B * H,),
        in_specs=[
            pl.BlockSpec((1, KV, D), lambda i, idx_ref: (i, 0, 0)),
            pl.BlockSpec((1, Q, D), lambda i, idx_ref: (i, 0, 0)),
        ],
        out_specs=pl.BlockSpec((1, KV, D), lambda i, idx_ref: (i, 0, 0)),
    )
    out3 = pl.pallas_call(
        _copy_scatter_kernel,
        grid_spec=grid_spec,
        out_shape=jax.ShapeDtypeStruct((B * H, KV, D), prev.dtype),
    )(idx, prev3, cur3)
    return out3.reshape(B, H, KV, D)
